# Initial kernel scaffold; baseline (speedup 1.0000x reference)
import jax
import jax.numpy as jnp
from jax.experimental import pallas as pl


def kernel(x, numerical_feature, W, b, t0, t1, t2, t3, t4, t5, t6, t7, t8, t9, t10, t11, t12, t13, t14):
    raise NotImplementedError("write your pallas kernel here")


# speedup vs baseline: 27.5303x; 27.5303x over previous
import functools

import jax
import jax.numpy as jnp
from jax import lax
from jax.experimental import pallas as pl
from jax.experimental.pallas import tpu as pltpu
from jax.experimental.pallas import tpu_sc as plsc

L = 16
NC = 2
NS = 16
NW = NC * NS
DMAX = 8


def _cvr_body(sizes, n_feat, n_num, cpad, bpw,
              xT_hbm, numT_hbm, tT_hbm, wT_hbm, wn_hbm, out_hbm,
              x_v, num_v, tt_v, wt_v, wn_v, cseg_v, c_v, acc_v, shc):
    cid = lax.axis_index("c")
    sid = lax.axis_index("s")
    wid = sid * NC + cid
    base = wid * bpw
    cseg = cpad // NS
    coff = sid * cseg

    pltpu.sync_copy(tT_hbm.at[:, pl.ds(coff, cseg)], tt_v)
    pltpu.sync_copy(wT_hbm.at[:, pl.ds(coff, cseg)], wt_v)
    for cc in range(cseg // L):
        sl = pl.ds(cc * L, L)
        acc = tt_v[0, sl] * wt_v[0, sl]
        for d in range(1, DMAX):
            acc = acc + tt_v[d, sl] * wt_v[d, sl]
        cseg_v[sl] = acc
    pltpu.sync_copy(cseg_v, shc.at[pl.ds(coff, cseg)])
    plsc.subcore_barrier()
    pltpu.sync_copy(shc, c_v)

    pltpu.sync_copy(xT_hbm.at[:, pl.ds(base, bpw)], x_v)
    pltpu.sync_copy(numT_hbm.at[:, pl.ds(base, bpw)], num_v)
    pltpu.sync_copy(wn_hbm, wn_v)
    wnv = wn_v[:]

    offs = []
    off = 0
    for n, _ in sizes:
        offs.append(off)
        off += n

    for ch in range(bpw // L):
        sl = pl.ds(ch * L, L)
        acc = num_v[0, sl] * wnv[0]
        for k in range(1, n_num + 1):
            acc = acc + num_v[k, sl] * wnv[k]
        for i in range(n_feat):
            idx = x_v[i, sl] + offs[i]
            acc = acc + plsc.load_gather(c_v, [idx])
        acc_v[sl] = acc
    pltpu.sync_copy(acc_v, out_hbm.at[pl.ds(base, bpw)])


def kernel(x, numerical_feature, W, b,
           t0, t1, t2, t3, t4, t5, t6, t7, t8, t9, t10, t11, t12, t13, t14):
    tables = [t0, t1, t2, t3, t4, t5, t6, t7, t8, t9, t10, t11, t12, t13, t14]
    sizes = tuple((t.shape[0], t.shape[1]) for t in tables)
    B = x.shape[0]
    n_feat = len(tables)
    n_num = numerical_feature.shape[1]
    ntot = sum(n for n, _ in sizes)
    cpad = NS * (-(-(-(-ntot // NS)) // 128) * 128)
    bpw = B // NW
    assert B % (NW * L) == 0

    tsegs, wsegs = [], []
    woff = 0
    for t, (n, d) in zip(tables, sizes):
        tsegs.append(jnp.pad(t, ((0, 0), (0, DMAX - d))))
        wrow = jnp.pad(W[woff:woff + d, 0], (0, DMAX - d))
        wsegs.append(jnp.broadcast_to(wrow, (n, DMAX)))
        woff += d
    tT = jnp.pad(jnp.concatenate(tsegs, 0), ((0, cpad - ntot), (0, 0))).T
    wT = jnp.pad(jnp.concatenate(wsegs, 0), ((0, cpad - ntot), (0, 0))).T
    xT = x.astype(jnp.int32).T
    numT = jnp.concatenate(
        [numerical_feature.T, jnp.ones((1, B), jnp.float32)], 0)
    wn = jnp.pad(jnp.concatenate([W[woff:, 0], b]), (0, L - n_num - 1))

    body = functools.partial(_cvr_body, sizes, n_feat, n_num, cpad, bpw)
    run = pl.kernel(
        body,
        out_type=jax.ShapeDtypeStruct((B,), jnp.float32),
        mesh=plsc.VectorSubcoreMesh(core_axis_name="c", subcore_axis_name="s",
                                    num_cores=NC, num_subcores=NS),
        compiler_params=pltpu.CompilerParams(needs_layout_passes=False),
        scratch_types=[
            pltpu.VMEM((n_feat, bpw), jnp.int32),
            pltpu.VMEM((n_num + 1, bpw), jnp.float32),
            pltpu.VMEM((DMAX, cpad // NS), jnp.float32),
            pltpu.VMEM((DMAX, cpad // NS), jnp.float32),
            pltpu.VMEM((L,), jnp.float32),
            pltpu.VMEM((cpad // NS,), jnp.float32),
            pltpu.VMEM((cpad,), jnp.float32),
            pltpu.VMEM((bpw,), jnp.float32),
            pltpu.VMEM_SHARED((cpad,), jnp.float32),
        ],
    )
    out = run(xT, numT, tT, wT, wn)
    return out.reshape(B, 1)


# speedup vs baseline: 30.1752x; 1.0961x over previous
import functools

import jax
import jax.numpy as jnp
from jax import lax
from jax.experimental import pallas as pl
from jax.experimental.pallas import tpu as pltpu
from jax.experimental.pallas import tpu_sc as plsc

L = 16
NC = 2
NS = 16
NW = NC * NS
DMAX = 8


def _cvr_body(sizes, n_feat, n_num, cpad, bpw,
              xT_hbm, numT_hbm, tT_hbm, wT_hbm, wn_hbm, out_hbm,
              x_v, num_v, tt_v, wt_v, wn_v, cseg_v, c_v, acc_v, shc,
              sem_fold, sem_in):
    cid = lax.axis_index("c")
    sid = lax.axis_index("s")
    wid = sid * NC + cid
    base = wid * bpw
    cseg = cpad // NS
    coff = sid * cseg

    h_t = pltpu.async_copy(tT_hbm.at[:, pl.ds(coff, cseg)], tt_v, sem_fold)
    h_w = pltpu.async_copy(wT_hbm.at[:, pl.ds(coff, cseg)], wt_v, sem_fold)
    h_x = pltpu.async_copy(xT_hbm.at[:, pl.ds(base, bpw)], x_v, sem_in)
    h_n = pltpu.async_copy(numT_hbm.at[:, pl.ds(base, bpw)], num_v, sem_in)
    h_wn = pltpu.async_copy(wn_hbm, wn_v, sem_in)

    h_t.wait()
    h_w.wait()
    for cc in range(cseg // L):
        sl = pl.ds(cc * L, L)
        acc = tt_v[0, sl] * wt_v[0, sl]
        for d in range(1, DMAX):
            acc = acc + tt_v[d, sl] * wt_v[d, sl]
        cseg_v[sl] = acc
    pltpu.sync_copy(cseg_v, shc.at[pl.ds(coff, cseg)])
    plsc.subcore_barrier()
    pltpu.sync_copy(shc, c_v)

    h_x.wait()
    h_n.wait()
    h_wn.wait()
    wnv = wn_v[:]

    offs = []
    off = 0
    for n, _ in sizes:
        offs.append(off)
        off += n

    for ch in range(bpw // L):
        sl = pl.ds(ch * L, L)
        acc = num_v[0, sl] * wnv[0]
        for k in range(1, n_num + 1):
            acc = acc + num_v[k, sl] * wnv[k]
        for i in range(n_feat):
            idx = x_v[i, sl] + offs[i]
            acc = acc + plsc.load_gather(c_v, [idx])
        acc_v[sl] = acc
    pltpu.sync_copy(acc_v, out_hbm.at[pl.ds(base, bpw)])


def kernel(x, numerical_feature, W, b,
           t0, t1, t2, t3, t4, t5, t6, t7, t8, t9, t10, t11, t12, t13, t14):
    tables = [t0, t1, t2, t3, t4, t5, t6, t7, t8, t9, t10, t11, t12, t13, t14]
    sizes = tuple((t.shape[0], t.shape[1]) for t in tables)
    B = x.shape[0]
    n_feat = len(tables)
    n_num = numerical_feature.shape[1]
    ntot = sum(n for n, _ in sizes)
    cpad = NS * (-(-(-(-ntot // NS)) // 128) * 128)
    bpw = B // NW
    assert B % (NW * L) == 0

    tsegs, wsegs = [], []
    woff = 0
    for t, (n, d) in zip(tables, sizes):
        tsegs.append(jnp.pad(t, ((0, 0), (0, DMAX - d))))
        wrow = jnp.pad(W[woff:woff + d, 0], (0, DMAX - d))
        wsegs.append(jnp.broadcast_to(wrow, (n, DMAX)))
        woff += d
    tT = jnp.pad(jnp.concatenate(tsegs, 0), ((0, cpad - ntot), (0, 0))).T
    wT = jnp.pad(jnp.concatenate(wsegs, 0), ((0, cpad - ntot), (0, 0))).T
    xT = x.astype(jnp.int32).T
    numT = jnp.concatenate(
        [numerical_feature.T, jnp.ones((1, B), jnp.float32)], 0)
    wn = jnp.pad(jnp.concatenate([W[woff:, 0], b]), (0, L - n_num - 1))

    body = functools.partial(_cvr_body, sizes, n_feat, n_num, cpad, bpw)
    run = pl.kernel(
        body,
        out_type=jax.ShapeDtypeStruct((B,), jnp.float32),
        mesh=plsc.VectorSubcoreMesh(core_axis_name="c", subcore_axis_name="s",
                                    num_cores=NC, num_subcores=NS),
        compiler_params=pltpu.CompilerParams(needs_layout_passes=False),
        scratch_types=[
            pltpu.VMEM((n_feat, bpw), jnp.int32),
            pltpu.VMEM((n_num + 1, bpw), jnp.float32),
            pltpu.VMEM((DMAX, cpad // NS), jnp.float32),
            pltpu.VMEM((DMAX, cpad // NS), jnp.float32),
            pltpu.VMEM((L,), jnp.float32),
            pltpu.VMEM((cpad // NS,), jnp.float32),
            pltpu.VMEM((cpad,), jnp.float32),
            pltpu.VMEM((bpw,), jnp.float32),
            pltpu.VMEM_SHARED((cpad,), jnp.float32),
            pltpu.SemaphoreType.DMA,
            pltpu.SemaphoreType.DMA,
        ],
    )
    out = run(xT, numT, tT, wT, wn)
    return out.reshape(B, 1)


# speedup vs baseline: 31.3024x; 1.0374x over previous
import functools

import jax
import jax.numpy as jnp
from jax import lax
from jax.experimental import pallas as pl
from jax.experimental.pallas import tpu as pltpu
from jax.experimental.pallas import tpu_sc as plsc

L = 16
NC = 2
NS = 16
NW = NC * NS
DMAX = 8


def _cvr_body(sizes, n_feat, n_num, cpad, bpw,
              xT_hbm, numT_hbm, tT_hbm, wT_hbm, wn_hbm, out_hbm,
              x_v, num_v, tt_v, wt_v, wn_v, cseg_v, c_v, acc_v, shc,
              sem_fold, sem_in):
    cid = lax.axis_index("c")
    sid = lax.axis_index("s")
    wid = sid * NC + cid
    base = wid * bpw
    cseg = cpad // NS
    coff = sid * cseg

    h_t = pltpu.async_copy(tT_hbm.at[:, pl.ds(coff, cseg)], tt_v, sem_fold)
    h_w = pltpu.async_copy(wT_hbm.at[:, pl.ds(coff, cseg)], wt_v, sem_fold)
    h_x = pltpu.async_copy(xT_hbm.at[:, pl.ds(base, bpw)], x_v, sem_in)
    h_n = pltpu.async_copy(numT_hbm.at[:, pl.ds(base, bpw)], num_v, sem_in)
    h_wn = pltpu.async_copy(wn_hbm, wn_v, sem_in)

    h_t.wait()
    h_w.wait()
    for cc in range(cseg // L):
        sl = pl.ds(cc * L, L)
        acc = tt_v[0, sl] * wt_v[0, sl]
        for d in range(1, DMAX):
            acc = acc + tt_v[d, sl] * wt_v[d, sl]
        cseg_v[sl] = acc
    pltpu.sync_copy(cseg_v, shc.at[pl.ds(coff, cseg)])
    plsc.subcore_barrier()
    pltpu.sync_copy(shc, c_v)

    h_x.wait()
    h_n.wait()
    h_wn.wait()
    wnv = wn_v[:]

    offs = []
    off = 0
    for n, _ in sizes:
        offs.append(off)
        off += n

    def chunk(ch, carry):
        sl = pl.ds(ch * L, L)
        acc = num_v[0, sl] * wnv[0]
        for k in range(1, n_num + 1):
            acc = acc + num_v[k, sl] * wnv[k]
        for i in range(n_feat):
            idx = x_v[i, sl] + offs[i]
            acc = acc + plsc.load_gather(c_v, [idx])
        acc_v[sl] = acc
        return carry

    lax.fori_loop(0, bpw // L, chunk, 0)
    pltpu.sync_copy(acc_v, out_hbm.at[pl.ds(base, bpw)])


def kernel(x, numerical_feature, W, b,
           t0, t1, t2, t3, t4, t5, t6, t7, t8, t9, t10, t11, t12, t13, t14):
    tables = [t0, t1, t2, t3, t4, t5, t6, t7, t8, t9, t10, t11, t12, t13, t14]
    sizes = tuple((t.shape[0], t.shape[1]) for t in tables)
    B = x.shape[0]
    n_feat = len(tables)
    n_num = numerical_feature.shape[1]
    ntot = sum(n for n, _ in sizes)
    cpad = NS * (-(-(-(-ntot // NS)) // 128) * 128)
    bpw = B // NW
    assert B % (NW * L) == 0

    tsegs, wsegs = [], []
    woff = 0
    for t, (n, d) in zip(tables, sizes):
        tsegs.append(jnp.pad(t, ((0, 0), (0, DMAX - d))))
        wrow = jnp.pad(W[woff:woff + d, 0], (0, DMAX - d))
        wsegs.append(jnp.broadcast_to(wrow, (n, DMAX)))
        woff += d
    tT = jnp.pad(jnp.concatenate(tsegs, 0), ((0, cpad - ntot), (0, 0))).T
    wT = jnp.pad(jnp.concatenate(wsegs, 0), ((0, cpad - ntot), (0, 0))).T
    xT = x.astype(jnp.int32).T
    numT = jnp.concatenate(
        [numerical_feature.T, jnp.ones((1, B), jnp.float32)], 0)
    wn = jnp.pad(jnp.concatenate([W[woff:, 0], b]), (0, L - n_num - 1))

    body = functools.partial(_cvr_body, sizes, n_feat, n_num, cpad, bpw)
    run = pl.kernel(
        body,
        out_type=jax.ShapeDtypeStruct((B,), jnp.float32),
        mesh=plsc.VectorSubcoreMesh(core_axis_name="c", subcore_axis_name="s",
                                    num_cores=NC, num_subcores=NS),
        compiler_params=pltpu.CompilerParams(needs_layout_passes=False),
        scratch_types=[
            pltpu.VMEM((n_feat, bpw), jnp.int32),
            pltpu.VMEM((n_num + 1, bpw), jnp.float32),
            pltpu.VMEM((DMAX, cpad // NS), jnp.float32),
            pltpu.VMEM((DMAX, cpad // NS), jnp.float32),
            pltpu.VMEM((L,), jnp.float32),
            pltpu.VMEM((cpad // NS,), jnp.float32),
            pltpu.VMEM((cpad,), jnp.float32),
            pltpu.VMEM((bpw,), jnp.float32),
            pltpu.VMEM_SHARED((cpad,), jnp.float32),
            pltpu.SemaphoreType.DMA,
            pltpu.SemaphoreType.DMA,
        ],
    )
    out = run(xT, numT, tT, wT, wn)
    return out.reshape(B, 1)
